# baseline (speedup 1.0000x reference)
import jax
import jax.numpy as jnp
from jax import lax
from jax.experimental import pallas as pl
from jax.experimental.pallas import tpu as pltpu
from jax.experimental.pallas import tpu_sc as plsc

N = 10000
D = 128
DH = D // 2
D2 = 16
NC, NS = 2, 16
NW = NC * NS
K = 512
KC = 1024
NP = 10240
RPT = NP // NS
ZB = 128


def _make_sc_agg(d, feat_split, with_deg, eb, k):
    mesh = plsc.VectorSubcoreMesh(
        core_axis_name="c", subcore_axis_name="s", num_cores=NC, num_subcores=NS)

    def body(table_hbm, src_hbm, dst_hbm, *rest):
        if with_deg:
            (acc_out, deg_out, src_v, dst_v, rows_v, zrow_v, ones_v, dvec_v,
             acc_sh, deg_sh, sem0) = rest
        else:
            (acc_out, src_v, dst_v, rows_v, zrow_v, acc_sh, sem0) = rest
        cid = lax.axis_index("c")
        sid = lax.axis_index("s")
        zv = jnp.zeros((16,), jnp.float32)
        ov = jnp.ones((16,), jnp.float32)

        if feat_split:
            pltpu.sync_copy(src_hbm.at[cid * NS + sid], src_v)
            pltpu.sync_copy(dst_hbm.at[sid], dst_v)
        else:
            wid = sid * NC + cid
            pltpu.sync_copy(src_hbm.at[wid], src_v)
            pltpu.sync_copy(dst_hbm.at[wid], dst_v)

        def _zrow(i, c):
            zrow_v[i // (d // 16), pl.ds((i % (d // 16)) * 16, 16)] = zv
            return c
        lax.fori_loop(0, (ZB * d) // 16, _zrow, 0)
        if with_deg:
            def _ones(i, c):
                ones_v[pl.ds(i * 16, 16)] = ov
                return c
            lax.fori_loop(0, k // 16, _ones, 0)
            def _zvec(i, c):
                dvec_v[pl.ds(i * 16, 16)] = zv
                return c
            lax.fori_loop(0, RPT // 16, _zvec, 0)

        base = sid * RPT
        def _zacc(i, c):
            pltpu.sync_copy(zrow_v, acc_sh.at[pl.ds(base + i * ZB, ZB)])
            return c
        lax.fori_loop(0, RPT // ZB, _zacc, 0)
        if with_deg:
            pltpu.sync_copy(dvec_v, deg_sh.at[pl.ds(base, RPT)])
        plsc.subcore_barrier()

        def _group(g, c):
            pltpu.async_copy(table_hbm.at[src_v.at[g]], rows_v, sem0).wait()
            pltpu.sync_copy(rows_v, acc_sh.at[dst_v.at[g]], add=True)
            if with_deg:
                own = (cid == 0) == (g < eb // 2)
                @pl.when(own)
                def _deg():
                    pltpu.sync_copy(ones_v, deg_sh.at[dst_v.at[g]], add=True)
            return c
        lax.fori_loop(0, eb, _group, 0)
        plsc.subcore_barrier()

        sl = pl.ds(base, RPT)
        pltpu.sync_copy(acc_sh.at[sl], acc_out.at[cid, sl])
        if with_deg:
            pltpu.sync_copy(deg_sh.at[pl.ds(base, RPT)], dvec_v)
            pltpu.sync_copy(dvec_v, deg_out.at[cid, pl.ds(base, RPT)])

    out_type = [jax.ShapeDtypeStruct((NC, NP, d), jnp.float32)]
    if with_deg:
        out_type.append(jax.ShapeDtypeStruct((NC, NP), jnp.float32))
    sc = [
        pltpu.VMEM((eb, k), jnp.int32),
        pltpu.VMEM((eb, k), jnp.int32),
        pltpu.VMEM((k, d), jnp.float32),
        pltpu.VMEM((ZB, d), jnp.float32),
    ]
    if with_deg:
        sc += [
            pltpu.VMEM((k,), jnp.float32),
            pltpu.VMEM((RPT,), jnp.float32),
        ]
    sc += [pltpu.VMEM_SHARED((NP, d), jnp.float32)]
    if with_deg:
        sc += [pltpu.VMEM_SHARED((NP,), jnp.float32)]
    sc += [pltpu.SemaphoreType.DMA]
    return pl.kernel(
        body,
        out_type=tuple(out_type) if with_deg else out_type[0],
        mesh=mesh,
        scratch_types=sc,
        compiler_params=pltpu.CompilerParams(use_tc_tiling_on_sc=False),
    )


def _tc_layer1(x_in, acc, deg_t, W1_l, W1_r, b1, W2_lp, W2_rp):
    br = 1000
    grid = (N // br,)

    def body(acc_ref, deg_ref, x_ref, wl_ref, wr_ref, b1_ref, w2l_ref,
             w2r_ref, p_ref, q_ref):
        deg = deg_ref[:, 0:1] + deg_ref[:, 1:2]
        inv = 1.0 / jnp.maximum(deg, 1.0)
        agg = jnp.concatenate([acc_ref[0], acc_ref[1]], axis=1) * inv
        h = jnp.dot(agg, wl_ref[...], preferred_element_type=jnp.float32)
        h += jnp.dot(x_ref[...], wr_ref[...], preferred_element_type=jnp.float32)
        h += b1_ref[...]
        h = jnp.maximum(h, 0.0)
        p_ref[...] = jnp.dot(h, w2l_ref[...], preferred_element_type=jnp.float32)
        q_ref[...] = jnp.dot(h, w2r_ref[...], preferred_element_type=jnp.float32)

    return pl.pallas_call(
        body,
        grid=grid,
        in_specs=[
            pl.BlockSpec((NC, br, DH), lambda i: (0, i, 0)),
            pl.BlockSpec((br, NC), lambda i: (i, 0)),
            pl.BlockSpec((br, D), lambda i: (i, 0)),
            pl.BlockSpec((D, D), lambda i: (0, 0)),
            pl.BlockSpec((D, D), lambda i: (0, 0)),
            pl.BlockSpec((1, D), lambda i: (0, 0)),
            pl.BlockSpec((D, D2), lambda i: (0, 0)),
            pl.BlockSpec((D, D2), lambda i: (0, 0)),
        ],
        out_specs=[
            pl.BlockSpec((br, D2), lambda i: (i, 0)),
            pl.BlockSpec((br, D2), lambda i: (i, 0)),
        ],
        out_shape=[
            jax.ShapeDtypeStruct((N, D2), jnp.float32),
            jax.ShapeDtypeStruct((N, D2), jnp.float32),
        ],
    )(acc, deg_t, x_in, W1_l, W1_r, b1, W2_lp, W2_rp)


def _tc_combine(acc2, deg_t, q, b2p):
    br = 1000
    grid = (N // br,)

    def body(acc_ref, deg_ref, q_ref, b2_ref, o_ref):
        deg = deg_ref[:, 0:1] + deg_ref[:, 1:2]
        inv = 1.0 / jnp.maximum(deg, 1.0)
        o_ref[...] = (acc_ref[0] + acc_ref[1]) * inv + q_ref[...] + b2_ref[...]

    return pl.pallas_call(
        body,
        grid=grid,
        in_specs=[
            pl.BlockSpec((NC, br, D2), lambda i: (0, i, 0)),
            pl.BlockSpec((br, NC), lambda i: (i, 0)),
            pl.BlockSpec((br, D2), lambda i: (i, 0)),
            pl.BlockSpec((1, D2), lambda i: (0, 0)),
        ],
        out_specs=pl.BlockSpec((br, D2), lambda i: (i, 0)),
        out_shape=jax.ShapeDtypeStruct((N, D2), jnp.float32),
    )(acc2, deg_t, q, b2p)


def kernel(x, edge_index, W1_l, W1_r, b1, W2_l, W2_r, b2):
    e = edge_index.shape[1]
    ka, kc = K, KC
    quantum = NW * max(ka, kc)
    epad = ((e + quantum - 1) // quantum) * quantum
    eb_a = epad // (NS * ka)
    eb_c = epad // (NW * kc)
    npad_e = epad - e

    src = edge_index[0].astype(jnp.int32)
    dst = edge_index[1].astype(jnp.int32)
    pad_i = jnp.arange(npad_e, dtype=jnp.int32)
    src_p = jnp.concatenate([src, pad_i % N])
    dst_p = jnp.concatenate([dst, N + pad_i % (NP - N)])

    src_a0 = src_p.reshape(NS, eb_a, ka)
    src_a = jnp.concatenate([src_a0, src_a0 + NP], axis=0)
    dst_a = dst_p.reshape(NS, eb_a, ka)
    src_c = src_p.reshape(NW, eb_c, kc)
    dst_c = dst_p.reshape(NW, eb_c, kc)

    x2 = (jnp.zeros((2 * NP, DH), jnp.float32)
          .at[:N].set(x[:, :DH]).at[NP:NP + N].set(x[:, DH:]))
    w2l_p = jnp.zeros((D, D2), jnp.float32).at[:, :2].set(W2_l)
    w2r_p = jnp.zeros((D, D2), jnp.float32).at[:, :2].set(W2_r)
    b2_p = jnp.zeros((1, D2), jnp.float32).at[0, :2].set(b2)
    b1_r = b1.reshape(1, D)

    agg_l1 = _make_sc_agg(DH, feat_split=True, with_deg=True, eb=eb_a, k=ka)
    agg_l2 = _make_sc_agg(D2, feat_split=False, with_deg=False, eb=eb_c, k=kc)

    acc1, deg = agg_l1(x2, src_a, dst_a)
    deg_t = deg.T
    p, q = _tc_layer1(x, acc1, deg_t, W1_l, W1_r, b1_r, w2l_p, w2r_p)
    acc2 = agg_l2(p, src_c, dst_c)
    out = _tc_combine(acc2, deg_t, q, b2_p)
    return out[:, :2]
